# Initial kernel scaffold; baseline (speedup 1.0000x reference)
import jax
import jax.numpy as jnp
from jax.experimental import pallas as pl


def kernel(z, embedding):
    raise NotImplementedError("write your pallas kernel here")


# speedup vs baseline: 1.5891x; 1.5891x over previous
import functools

import jax
import jax.numpy as jnp
from jax import lax
from jax.experimental import pallas as pl
from jax.experimental.pallas import tpu as pltpu
from jax.experimental.pallas import tpu_sc as plsc

N_E = 1024
E_DIM = 256
BETA = 0.25
TOK = 8192
BLK = 512
GRID = TOK // BLK

_NC = 2
_NS = 16
_NW = _NC * _NS
_BPW = TOK // _NW


def _dist_body(z_ref, e_ref, d_ref, idx_ref, loss_ref, acc_ref):
    i = pl.program_id(0)
    z = z_ref[...]
    e = e_ref[...]
    zn = jnp.sum(z * z, axis=1, keepdims=True)
    en = jnp.sum(e * e, axis=1)[None, :]
    cross = lax.dot_general(z.astype(jnp.bfloat16), e.astype(jnp.bfloat16),
                            (((1,), (1,)), ((), ())),
                            preferred_element_type=jnp.float32)
    d = (zn + en) - 2.0 * cross
    d_ref[...] = d
    dmin = jnp.min(d, axis=1, keepdims=True)
    col = lax.broadcasted_iota(jnp.int32, d.shape, 1)
    idx = jnp.min(jnp.where(d == dmin, col, N_E), axis=1)
    idx_ref[0, 0, :] = idx

    @pl.when(i == 0)
    def _():
        acc_ref[0] = 0.0

    acc_ref[0] += jnp.sum(dmin[:, 0])
    loss_ref[...] = jnp.full((1, 1), acc_ref[0] * ((1.0 + BETA) / (TOK * E_DIM)),
                             jnp.float32)


_dist_call = pl.pallas_call(
    _dist_body,
    grid=(GRID,),
    in_specs=[
        pl.BlockSpec((BLK, E_DIM), lambda i: (i, 0)),
        pl.BlockSpec((N_E, E_DIM), lambda i: (0, 0)),
    ],
    out_specs=[
        pl.BlockSpec((BLK, N_E), lambda i: (i, 0)),
        pl.BlockSpec((1, 1, BLK), lambda i: (i, 0, 0)),
        pl.BlockSpec((1, 1), lambda i: (0, 0)),
    ],
    out_shape=[
        jax.ShapeDtypeStruct((TOK, N_E), jnp.float32),
        jax.ShapeDtypeStruct((GRID, 1, BLK), jnp.int32),
        jax.ShapeDtypeStruct((1, 1), jnp.float32),
    ],
    scratch_shapes=[pltpu.SMEM((1,), jnp.float32)],
)


def _gather_body(table_hbm, idx_hbm, out_hbm, idx_v, rows_v, sem):
    wid = lax.axis_index("s") * _NC + lax.axis_index("c")
    base = wid * _BPW
    pltpu.sync_copy(idx_hbm.at[pl.ds(base, _BPW)], idx_v)
    pltpu.async_copy(table_hbm.at[idx_v], rows_v, sem).wait()
    pltpu.sync_copy(rows_v, out_hbm.at[pl.ds(base, _BPW)])


@functools.cache
def _gather_call():
    return pl.kernel(
        _gather_body,
        out_type=jax.ShapeDtypeStruct((TOK, E_DIM), jnp.float32),
        mesh=plsc.VectorSubcoreMesh(core_axis_name="c", subcore_axis_name="s",
                                    num_cores=_NC, num_subcores=_NS),
        scratch_types=[
            pltpu.VMEM((_BPW,), jnp.int32),
            pltpu.VMEM((_BPW, E_DIM), jnp.float32),
            pltpu.SemaphoreType.DMA,
        ],
    )


def kernel(z, embedding):
    B, C, H, W = z.shape
    z_flat = jnp.transpose(z, (0, 2, 3, 1)).reshape(TOK, E_DIM)
    d, idx3, loss2 = _dist_call(z_flat, embedding)
    idx = idx3.reshape(TOK)
    z_q_flat = _gather_call()(embedding, idx)
    z_q_out = jnp.transpose(z_q_flat.reshape(B, H, W, C), (0, 3, 1, 2))
    return (z_q_out, loss2[0, 0], idx, d.reshape(B, H, W, N_E))
